# baseline (speedup 1.0000x reference)
import functools

import jax
import jax.numpy as jnp
from jax import lax
from jax.experimental import pallas as pl
from jax.experimental.pallas import tpu as pltpu
from jax.experimental.pallas import tpu_sc as plsc

NUM_ENT = 100000
NUM_REL = 1000
S_DIM = 64
T_DIM = 64
MARGIN = 10.0
B = 16384

NC = 2
NS = 16
NW = NC * NS
EW = B // NW
CHUNK = 128
NCH = EW // CHUNK
NSLOT = 42

def _sc_mesh():
    return plsc.VectorSubcoreMesh(core_axis_name="c", subcore_axis_name="s",
                                  num_cores=NC, num_subcores=NS)


def _sc_gather_body(ih, it, ir, *rest):
    tabs = rest[:22]
    out = rest[22]
    ihv, itv, irv, buf, sem0, sem1 = rest[23:]
    sems = (sem0, sem1)

    wid = lax.axis_index("s") * NC + lax.axis_index("c")

    pltpu.sync_copy(ih.at[wid], ihv)
    pltpu.sync_copy(it.at[wid], itv)
    pltpu.sync_copy(ir.at[wid], irv)

    gathers = [(0, ihv), (0, itv), (1, itv), (1, ihv)]
    gathers += [(4 + i, ihv) for i in range(9)]
    gathers += [(13 + i, itv) for i in range(9)]
    gathers += [(4 + i, itv) for i in range(9)]
    gathers += [(13 + i, ihv) for i in range(9)]
    gathers += [(2, irv), (3, irv)]

    def chunk_body(c, carry):
        rowbase = wid * EW + c * CHUNK

        def start(k):
            ti, idxv = gathers[k]
            return pltpu.async_copy(tabs[ti].at[idxv.at[c]], buf.at[k % 2],
                                    sems[k % 2])

        cp = start(0)
        for k in range(1, NSLOT):
            cp_next = start(k)
            cp.wait()
            pltpu.sync_copy(buf.at[(k - 1) % 2],
                            out.at[k - 1, pl.ds(rowbase, CHUNK)])
            cp = cp_next
        cp.wait()
        pltpu.sync_copy(buf.at[(NSLOT - 1) % 2],
                        out.at[NSLOT - 1, pl.ds(rowbase, CHUNK)])
        return carry

    lax.fori_loop(0, NCH, chunk_body, 0)


@functools.partial(jax.jit, static_argnames=())
def _sc_gather(ih, it, ir, tables):
    f = pl.kernel(
        _sc_gather_body,
        out_type=jax.ShapeDtypeStruct((NSLOT, B, S_DIM), jnp.float32),
        mesh=_sc_mesh(),
        scratch_types=[
            pltpu.VMEM((NCH, CHUNK), jnp.int32),
            pltpu.VMEM((NCH, CHUNK), jnp.int32),
            pltpu.VMEM((NCH, CHUNK), jnp.int32),
            pltpu.VMEM((2, CHUNK, S_DIM), jnp.float32),
            pltpu.SemaphoreType.DMA,
            pltpu.SemaphoreType.DMA,
        ],
        compiler_params=pltpu.CompilerParams(use_tc_tiling_on_sc=False),
    )
    return f(ih, it, ir, *tables)


BT = 256


def _tc_score_body(y_ref, m_ref, d_ref, g_ref, o_ref):
    yy = y_ref[:]
    mm = m_ref[:]
    dd = d_ref[:]

    def S(k):
        return g_ref[k]

    def temb(base):
        e = S(base + 2) * jnp.sin(S(base) * yy + S(base + 1))
        e += S(base + 5) * jnp.sin(S(base + 3) * mm + S(base + 4))
        e += S(base + 8) * jnp.sin(S(base + 6) * dd + S(base + 7))
        return e

    h_re_s, t_re_s, h_im_s, t_im_s = S(0), S(1), S(2), S(3)
    h_re_t = temb(4)
    t_re_t = temb(13)
    h_im_t = temb(22)
    t_im_t = temb(31)
    r_s, r_t = S(40), S(41)

    def part(h_re, h_im, t_re, t_im, r):
        cr = jnp.cos(r)
        sr = jnp.sin(r)
        re = h_re * cr - h_im * sr - t_re
        im = h_re * sr + h_im * cr - t_im
        return jnp.sum(jnp.sqrt(re * re + im * im), axis=1)

    tot = part(h_re_s, h_im_s, t_re_s, t_im_s, r_s)
    tot += part(h_re_t, h_im_t, t_re_t, t_im_t, r_t)
    o_ref[:] = MARGIN - tot


def _tc_score(years, months, days, g):
    grid = (B // BT,)
    return pl.pallas_call(
        _tc_score_body,
        grid=grid,
        in_specs=[
            pl.BlockSpec((BT, 1), lambda i: (i, 0)),
            pl.BlockSpec((BT, 1), lambda i: (i, 0)),
            pl.BlockSpec((BT, 1), lambda i: (i, 0)),
            pl.BlockSpec((NSLOT, BT, S_DIM), lambda i: (0, i, 0)),
        ],
        out_specs=pl.BlockSpec((BT,), lambda i: (i,)),
        out_shape=jax.ShapeDtypeStruct((B,), jnp.float32),
    )(years, months, days, g)


def kernel(heads, rels, tails, years, months, days, ent_embs_h, ent_embs_t,
           rel_embs, y_freq_h, y_freq_t, y_phi_h, y_phi_t, y_amps_h,
           y_amps_t, m_freq_h, m_freq_t, m_phi_h, m_phi_t, m_amps_h,
           m_amps_t, d_freq_h, d_freq_t, d_phi_h, d_phi_t, d_amps_h,
           d_amps_t):
    ih = heads.astype(jnp.int32).reshape(NW, NCH, CHUNK)
    it = tails.astype(jnp.int32).reshape(NW, NCH, CHUNK)
    ir = rels.astype(jnp.int32).reshape(NW, NCH, CHUNK)
    rel_s = rel_embs[:, :S_DIM]
    rel_t = rel_embs[:, S_DIM:]
    tables = (
        ent_embs_h, ent_embs_t, rel_s, rel_t,
        y_freq_h, y_phi_h, y_amps_h,
        m_freq_h, m_phi_h, m_amps_h,
        d_freq_h, d_phi_h, d_amps_h,
        y_freq_t, y_phi_t, y_amps_t,
        m_freq_t, m_phi_t, m_amps_t,
        d_freq_t, d_phi_t, d_amps_t,
    )
    g = _sc_gather(ih, it, ir, tables)
    return _tc_score(years.reshape(B, 1), months.reshape(B, 1),
                     days.reshape(B, 1), g)


# speedup vs baseline: 1.5357x; 1.5357x over previous
import jax
import jax.numpy as jnp
from jax import lax
from jax.experimental import pallas as pl
from jax.experimental.pallas import tpu as pltpu
from jax.experimental.pallas import tpu_sc as plsc

NUM_ENT = 100000
NUM_REL = 1000
MARGIN = 10.0
B = 16384
D = 128

NC = 2
NS = 16
NW = NC * NS
EW = B // NW
CHUNK = 128
NCH = EW // CHUNK
NPAIR = 10
NSLOT = 2 * NPAIR + 1


def _sc_mesh():
    return plsc.VectorSubcoreMesh(core_axis_name="c", subcore_axis_name="s",
                                  num_cores=NC, num_subcores=NS)


def _sc_gather_body(ih, it, ir, *rest):
    tabs = rest[:NPAIR + 1]
    out = rest[NPAIR + 1]
    ihv, itv, irv, buf, sem0, sem1 = rest[NPAIR + 2:]
    sems = (sem0, sem1)

    wid = lax.axis_index("s") * NC + lax.axis_index("c")

    pltpu.sync_copy(ih.at[wid], ihv)
    pltpu.sync_copy(it.at[wid], itv)
    pltpu.sync_copy(ir.at[wid], irv)

    gathers = ([(i, ihv) for i in range(NPAIR)]
               + [(i, itv) for i in range(NPAIR)]
               + [(NPAIR, irv)])

    def chunk_body(c, carry):
        rowbase = wid * EW + c * CHUNK

        def start(k):
            ti, idxv = gathers[k]
            return pltpu.async_copy(tabs[ti].at[idxv.at[c]], buf.at[k % 2],
                                    sems[k % 2])

        cp = start(0)
        for k in range(1, NSLOT):
            cp_next = start(k)
            cp.wait()
            pltpu.sync_copy(buf.at[(k - 1) % 2],
                            out.at[k - 1, pl.ds(rowbase, CHUNK)])
            cp = cp_next
        cp.wait()
        pltpu.sync_copy(buf.at[(NSLOT - 1) % 2],
                        out.at[NSLOT - 1, pl.ds(rowbase, CHUNK)])
        return carry

    lax.fori_loop(0, NCH, chunk_body, 0)


def _sc_gather(ih, it, ir, tables):
    f = pl.kernel(
        _sc_gather_body,
        out_type=jax.ShapeDtypeStruct((NSLOT, B, D), jnp.float32),
        mesh=_sc_mesh(),
        scratch_types=[
            pltpu.VMEM((NCH, CHUNK), jnp.int32),
            pltpu.VMEM((NCH, CHUNK), jnp.int32),
            pltpu.VMEM((NCH, CHUNK), jnp.int32),
            pltpu.VMEM((2, CHUNK, D), jnp.float32),
            pltpu.SemaphoreType.DMA,
            pltpu.SemaphoreType.DMA,
        ],
        compiler_params=pltpu.CompilerParams(use_tc_tiling_on_sc=False),
    )
    return f(ih, it, ir, *tables)


BT = 256


def _tc_score_body(y_ref, m_ref, d_ref, g_ref, o_ref):
    yy = y_ref[:]
    mm = m_ref[:]
    dd = d_ref[:]

    def S(k):
        return g_ref[k]

    def temb(b):
        e = S(b + 2) * jnp.sin(S(b) * yy + S(b + 1))
        e += S(b + 5) * jnp.sin(S(b + 3) * mm + S(b + 4))
        e += S(b + 8) * jnp.sin(S(b + 6) * dd + S(b + 7))
        return e

    a_h = S(0)
    a_t = S(10)
    t_h = temb(1)
    t_t = temb(11)
    r = S(20)
    cr = jnp.cos(r)
    sr = jnp.sin(r)

    def part(h_re, h_im, t_re, t_im, c, s):
        re = h_re * c - h_im * s - t_re
        im = h_re * s + h_im * c - t_im
        return jnp.sum(jnp.sqrt(re * re + im * im), axis=1)

    H = 64
    tot = part(a_h[:, :H], a_t[:, H:], a_t[:, :H], a_h[:, H:],
               cr[:, :H], sr[:, :H])
    tot += part(t_h[:, :H], t_t[:, :H], t_t[:, H:], t_h[:, H:],
                cr[:, H:], sr[:, H:])
    o_ref[:] = MARGIN - tot


def _tc_score(years, months, days, g):
    return pl.pallas_call(
        _tc_score_body,
        grid=(B // BT,),
        in_specs=[
            pl.BlockSpec((BT, 1), lambda i: (i, 0)),
            pl.BlockSpec((BT, 1), lambda i: (i, 0)),
            pl.BlockSpec((BT, 1), lambda i: (i, 0)),
            pl.BlockSpec((NSLOT, BT, D), lambda i: (0, i, 0)),
        ],
        out_specs=pl.BlockSpec((BT,), lambda i: (i,)),
        out_shape=jax.ShapeDtypeStruct((B,), jnp.float32),
    )(years, months, days, g)


def kernel(heads, rels, tails, years, months, days, ent_embs_h, ent_embs_t,
           rel_embs, y_freq_h, y_freq_t, y_phi_h, y_phi_t, y_amps_h,
           y_amps_t, m_freq_h, m_freq_t, m_phi_h, m_phi_t, m_amps_h,
           m_amps_t, d_freq_h, d_freq_t, d_phi_h, d_phi_t, d_amps_h,
           d_amps_t):
    ih = heads.astype(jnp.int32).reshape(NW, NCH, CHUNK)
    it = tails.astype(jnp.int32).reshape(NW, NCH, CHUNK)
    ir = rels.astype(jnp.int32).reshape(NW, NCH, CHUNK)

    def pack(a, b):
        return jnp.concatenate([a, b], axis=1)

    tables = (
        pack(ent_embs_h, ent_embs_t),
        pack(y_freq_h, y_freq_t), pack(y_phi_h, y_phi_t),
        pack(y_amps_h, y_amps_t),
        pack(m_freq_h, m_freq_t), pack(m_phi_h, m_phi_t),
        pack(m_amps_h, m_amps_t),
        pack(d_freq_h, d_freq_t), pack(d_phi_h, d_phi_t),
        pack(d_amps_h, d_amps_t),
        rel_embs,
    )
    g = _sc_gather(ih, it, ir, tables)
    return _tc_score(years.reshape(B, 1), months.reshape(B, 1),
                     days.reshape(B, 1), g)


# speedup vs baseline: 1.5382x; 1.0016x over previous
import jax
import jax.numpy as jnp
from jax import lax
from jax.experimental import pallas as pl
from jax.experimental.pallas import tpu as pltpu
from jax.experimental.pallas import tpu_sc as plsc

NUM_ENT = 100000
NUM_REL = 1000
MARGIN = 10.0
B = 16384
D = 128

NC = 2
NS = 16
NW = NC * NS
EW = B // NW
CHUNK = 128
NCH = EW // CHUNK
NPAIR = 10
NSLOT = 2 * NPAIR + 1


def _sc_mesh():
    return plsc.VectorSubcoreMesh(core_axis_name="c", subcore_axis_name="s",
                                  num_cores=NC, num_subcores=NS)


def _sc_gather_body(ih, it, ir, *rest):
    tabs = rest[:NPAIR + 1]
    out = rest[NPAIR + 1]
    ihv, itv, irv, buf, sem0, sem1 = rest[NPAIR + 2:]
    sems = (sem0, sem1)

    wid = lax.axis_index("s") * NC + lax.axis_index("c")

    pltpu.sync_copy(ih.at[wid], ihv)
    pltpu.sync_copy(it.at[wid], itv)
    pltpu.sync_copy(ir.at[wid], irv)

    gathers = ([(i, ihv) for i in range(NPAIR)]
               + [(i, itv) for i in range(NPAIR)]
               + [(NPAIR, irv)])

    def chunk_body(c, carry):
        rowbase = wid * EW + c * CHUNK

        def start(k):
            ti, idxv = gathers[k]
            return pltpu.async_copy(tabs[ti].at[idxv.at[c]], buf.at[k % 2],
                                    sems[k % 2])

        cp = start(0)
        for k in range(1, NSLOT):
            cp_next = start(k)
            cp.wait()
            pltpu.sync_copy(buf.at[(k - 1) % 2],
                            out.at[k - 1, pl.ds(rowbase, CHUNK)])
            cp = cp_next
        cp.wait()
        pltpu.sync_copy(buf.at[(NSLOT - 1) % 2],
                        out.at[NSLOT - 1, pl.ds(rowbase, CHUNK)])
        return carry

    lax.fori_loop(0, NCH, chunk_body, 0)


def _sc_gather(ih, it, ir, tables):
    f = pl.kernel(
        _sc_gather_body,
        out_type=jax.ShapeDtypeStruct((NSLOT, B, D), jnp.float32),
        mesh=_sc_mesh(),
        scratch_types=[
            pltpu.VMEM((NCH, CHUNK), jnp.int32),
            pltpu.VMEM((NCH, CHUNK), jnp.int32),
            pltpu.VMEM((NCH, CHUNK), jnp.int32),
            pltpu.VMEM((2, CHUNK, D), jnp.float32),
            pltpu.SemaphoreType.DMA,
            pltpu.SemaphoreType.DMA,
        ],
        compiler_params=pltpu.CompilerParams(use_tc_tiling_on_sc=False),
    )
    return f(ih, it, ir, *tables)


BT = 256


def _tc_score_body(y_ref, m_ref, d_ref, g_ref, o_ref):
    yy = y_ref[:]
    mm = m_ref[:]
    dd = d_ref[:]

    def S(k):
        return g_ref[k]

    def temb(b):
        e = S(b + 2) * jnp.sin(S(b) * yy + S(b + 1))
        e += S(b + 5) * jnp.sin(S(b + 3) * mm + S(b + 4))
        e += S(b + 8) * jnp.sin(S(b + 6) * dd + S(b + 7))
        return e

    a_h = S(0)
    a_t = S(10)
    t_h = temb(1)
    t_t = temb(11)
    r = S(20)
    cr = jnp.cos(r)
    sr = jnp.sin(r)

    def part(h_re, h_im, t_re, t_im, c, s):
        re = h_re * c - h_im * s - t_re
        im = h_re * s + h_im * c - t_im
        return jnp.sum(jnp.sqrt(re * re + im * im), axis=1)

    H = 64
    tot = part(a_h[:, :H], a_t[:, H:], a_t[:, :H], a_h[:, H:],
               cr[:, :H], sr[:, :H])
    tot += part(t_h[:, :H], t_t[:, :H], t_t[:, H:], t_h[:, H:],
                cr[:, H:], sr[:, H:])
    o_ref[:] = MARGIN - tot


def _tc_score(years, months, days, g):
    return pl.pallas_call(
        _tc_score_body,
        grid=(B // BT,),
        in_specs=[
            pl.BlockSpec((BT, 1), lambda i: (i, 0)),
            pl.BlockSpec((BT, 1), lambda i: (i, 0)),
            pl.BlockSpec((BT, 1), lambda i: (i, 0)),
            pl.BlockSpec((NSLOT, BT, D), lambda i: (0, i, 0)),
        ],
        out_specs=pl.BlockSpec((BT,), lambda i: (i,)),
        out_shape=jax.ShapeDtypeStruct((B,), jnp.float32),
    )(years, months, days, g)


def kernel(heads, rels, tails, years, months, days, ent_embs_h, ent_embs_t,
           rel_embs, y_freq_h, y_freq_t, y_phi_h, y_phi_t, y_amps_h,
           y_amps_t, m_freq_h, m_freq_t, m_phi_h, m_phi_t, m_amps_h,
           m_amps_t, d_freq_h, d_freq_t, d_phi_h, d_phi_t, d_amps_h,
           d_amps_t):
    ih = heads.astype(jnp.int32).reshape(NW, NCH, CHUNK)
    it = tails.astype(jnp.int32).reshape(NW, NCH, CHUNK)
    ir = rels.astype(jnp.int32).reshape(NW, NCH, CHUNK)

    def pack(a, b):
        return jnp.concatenate([a.T, b.T], axis=0).T

    tables = (
        pack(ent_embs_h, ent_embs_t),
        pack(y_freq_h, y_freq_t), pack(y_phi_h, y_phi_t),
        pack(y_amps_h, y_amps_t),
        pack(m_freq_h, m_freq_t), pack(m_phi_h, m_phi_t),
        pack(m_amps_h, m_amps_t),
        pack(d_freq_h, d_freq_t), pack(d_phi_h, d_phi_t),
        pack(d_amps_h, d_amps_t),
        rel_embs,
    )
    g = _sc_gather(ih, it, ir, tables)
    return _tc_score(years.reshape(B, 1), months.reshape(B, 1),
                     days.reshape(B, 1), g)
